# baseline (speedup 1.0000x reference)
import functools

import jax
import jax.numpy as jnp
from jax import lax
from jax.experimental import pallas as pl
from jax.experimental.pallas import tpu as pltpu
from jax.experimental.pallas import tpu_sc as plsc

NC, NS, L = 2, 16, 16
NW = NC * NS
D = 64
SCALE = 8.0
CHUNK = 256
NBUF = 2


@functools.partial(jax.jit, static_argnames=("B",))
def _sc_lookup(idx_flat, table, B):
    b_per_w = B // NW
    n_chunks = b_per_w // CHUNK
    mesh = plsc.VectorSubcoreMesh(core_axis_name="c", subcore_axis_name="s")

    @functools.partial(
        pl.kernel,
        out_type=jax.ShapeDtypeStruct((B, D), jnp.float32),
        mesh=mesh,
        scratch_types=[
            pltpu.VMEM((b_per_w,), jnp.int32),
            pltpu.VMEM((CHUNK, D), jnp.float32),
            pltpu.VMEM_SHARED((NS, CHUNK, D), jnp.float32),
            pltpu.SemaphoreType.DMA,
            pltpu.SemaphoreType.DMA,
        ],
    )
    def k(idx_hbm, table_hbm, out_hbm, idx_v, rows_v, shared, sem_g, sem_o):
        cid = lax.axis_index("c")
        sid = lax.axis_index("s")
        wid = sid * NC + cid
        base = wid * b_per_w

        pltpu.sync_copy(
            idx_hbm.at[pl.ds(pl.multiple_of(base, 256), b_per_w)], idx_v)

        def chunk_body(c, carry):
            def fire(gg, carry2):
                g16 = idx_v[pl.ds(c * CHUNK + gg * L, L)]
                for r in range(L):
                    pltpu.async_copy(
                        table_hbm.at[pl.ds(g16[r], 1)],
                        shared.at[sid, pl.ds(gg * L + r, 1)],
                        sem_g,
                    )
                return carry2

            lax.fori_loop(0, CHUNK // L, fire, 0, unroll=False)
            pltpu.make_async_copy(
                out_hbm.at[pl.ds(0, CHUNK)], shared.at[sid], sem_g
            ).wait()
            pltpu.sync_copy(shared.at[sid], rows_v)

            def grp_body(gg, carry2):
                g16 = idx_v[pl.ds(c * CHUNK + gg * L, L)]
                m16 = jnp.where(g16 != 0, SCALE, 0.0).astype(jnp.float32)
                for r in range(L):
                    m = m16.at[jnp.full((L,), r, jnp.int32)].get(
                        mode="promise_in_bounds")
                    row = gg * L + r
                    for kk in range(D // L):
                        v = rows_v[row, pl.ds(kk * L, L)]
                        rows_v[row, pl.ds(kk * L, L)] = v * m
                return carry2

            lax.fori_loop(0, CHUNK // L, grp_body, 0, unroll=False)
            pltpu.sync_copy(rows_v, shared.at[sid])
            pltpu.async_copy(
                shared.at[sid],
                out_hbm.at[pl.ds(
                    pl.multiple_of(base + c * CHUNK, 256), CHUNK)],
                sem_o,
            )
            pltpu.make_async_copy(
                shared.at[sid], out_hbm.at[pl.ds(0, CHUNK)], sem_o
            ).wait()
            return carry

        lax.fori_loop(0, n_chunks, chunk_body, 0, unroll=False)

    return k(idx_flat, table)


def kernel(inputs, shared_weights):
    B = inputs.size
    idx_flat = inputs.reshape(B).astype(jnp.int32)
    out = _sc_lookup(idx_flat, shared_weights, B)
    return out.reshape(inputs.shape + (D,))
